# baseline (speedup 1.0000x reference)
import jax
import jax.numpy as jnp
from jax.experimental import pallas as pl
from jax.experimental.pallas import tpu as pltpu

_DEPTH = 1000
_ROWS = 4096
_COLS = 26
_BR = 32


def _body(ids_ref, out_ref):
    ids = ids_ref[...]
    iota = jax.lax.broadcasted_iota(jnp.int32, (_BR, _DEPTH), 1)
    for j in range(_COLS):
        idj = ids[:, j][:, None]
        out_ref[:, j, :] = (idj == iota).astype(jnp.float32)


def kernel(inputs):
    ids = inputs.astype(jnp.int32)
    return pl.pallas_call(
        _body,
        grid=(_ROWS // _BR,),
        in_specs=[pl.BlockSpec((_BR, _COLS), lambda i: (i, 0))],
        out_specs=pl.BlockSpec((_BR, _COLS, _DEPTH), lambda i: (i, 0, 0)),
        out_shape=jax.ShapeDtypeStruct((_ROWS, _COLS, _DEPTH), jnp.float32),
        compiler_params=pltpu.CompilerParams(
            dimension_semantics=("parallel",),
        ),
    )(ids)


# speedup vs baseline: 5.1571x; 5.1571x over previous
import jax
import jax.numpy as jnp
from jax.experimental import pallas as pl
from jax.experimental.pallas import tpu as pltpu

_DEPTH = 1000
_ROWS = 4096
_COLS = 26
_D_BLK = 200
_B_BLK = 512


def _body(ids_ref, out_ref):
    d0 = pl.program_id(0) * _D_BLK
    iota = jax.lax.broadcasted_iota(jnp.int32, (_D_BLK, _B_BLK), 0) + d0
    ids = ids_ref[...]
    for j in range(_COLS):
        idj = ids[j, :][None, :]
        out_ref[j] = (iota == idj).astype(jnp.float32)


def kernel(inputs):
    ids_t = jnp.transpose(inputs.astype(jnp.int32), (1, 0))
    out_t = pl.pallas_call(
        _body,
        grid=(_DEPTH // _D_BLK, _ROWS // _B_BLK),
        in_specs=[pl.BlockSpec((_COLS, _B_BLK), lambda d, b: (0, b))],
        out_specs=pl.BlockSpec((_COLS, _D_BLK, _B_BLK), lambda d, b: (0, d, b)),
        out_shape=jax.ShapeDtypeStruct((_COLS, _DEPTH, _ROWS), jnp.float32),
        compiler_params=pltpu.CompilerParams(
            dimension_semantics=("parallel", "parallel"),
        ),
    )(ids_t)
    return jnp.transpose(out_t, (2, 0, 1))
